# baseline (speedup 1.0000x reference)
import functools

import jax
import jax.numpy as jnp
from jax import lax
from jax.experimental import pallas as pl
from jax.experimental.pallas import tpu as pltpu
from jax.experimental.pallas import tpu_sc as plsc

F = 39
FP = 40
K = 10
KP = 16
B = 16384
TOTAL = B * FP
NC, NS = 2, 16
NW = NC * NS
PER_W = TOTAL // NW
CH = 5120
NCH = PER_W // CH


@functools.cache
def _make_sc_gather():
    mesh = plsc.VectorSubcoreMesh(core_axis_name="c", subcore_axis_name="s")

    @functools.partial(
        pl.kernel,
        mesh=mesh,
        out_type=jax.ShapeDtypeStruct((TOTAL, KP), jnp.float32),
        scratch_types=[
            pltpu.VMEM((CH,), jnp.int32),
            pltpu.VMEM((CH, KP), jnp.float32),
            pltpu.SemaphoreType.DMA,
        ],
        compiler_params=pltpu.CompilerParams(use_tc_tiling_on_sc=False),
    )
    def _sc_gather(idx_hbm, tab_hbm, out_hbm, idx_v, rows_v, sem):
        wid = lax.axis_index("s") * NC + lax.axis_index("c")
        base = wid * PER_W

        def body(j, carry):
            off = base + j * CH
            pltpu.sync_copy(idx_hbm.at[pl.ds(off, CH)], idx_v)
            pltpu.async_copy(tab_hbm.at[idx_v], rows_v, sem).wait()
            pltpu.sync_copy(rows_v, out_hbm.at[pl.ds(off, CH)])
            return carry

        lax.fori_loop(0, NCH, body, 0)

    return _sc_gather


CN = 65536
CN8 = CN // 8
SH = CN8.bit_length() - 1


def _pack_body(vt_ref, wt_ref, out_ref):
    vt = vt_ref[...]
    wt = wt_ref[...]
    z = jnp.zeros((KP - K - 1, CN), jnp.float32)
    m = jnp.concatenate([vt, wt, z], axis=0)
    for q in range(CN // 1024):
        mq = jnp.concatenate(
            [m[:, CN8 * k + 128 * q: CN8 * k + 128 * q + 128]
             for k in range(8)], axis=0)
        out_ref[128 * q:128 * (q + 1), :] = mq.T


def _permute_idx(idx):
    o = idx & (CN - 1)
    return (idx & ~(CN - 1)) | ((o & (CN8 - 1)) << 3) | (o >> SH)


def _pack_table(v, w):
    fea = v.shape[0]
    grid = (fea + CN - 1) // CN
    out2d = pl.pallas_call(
        _pack_body,
        grid=(grid,),
        in_specs=[
            pl.BlockSpec((K, CN), lambda i: (0, i)),
            pl.BlockSpec((1, CN), lambda i: (0, i)),
        ],
        out_specs=pl.BlockSpec((CN * KP // 128, 128), lambda i: (i, 0)),
        out_shape=jax.ShapeDtypeStruct((grid * CN * KP // 128, 128), jnp.float32),
    )(v.T, w.T)
    return out2d.reshape(grid * CN, KP)


BB = 512


def _bdot(a, b):
    return jnp.dot(a.astype(jnp.bfloat16), b.astype(jnp.bfloat16),
                   preferred_element_type=jnp.float32)


def _tc_body(g_ref, s_ref, w0_ref, b0_ref, w1_ref, b1_ref,
             w2_ref, b2_ref, w3_ref, b3_ref, out_ref):
    g = g_ref[...]
    s = s_ref[...]
    sv = jnp.dot(g, s, preferred_element_type=jnp.float32)
    sv2 = jnp.dot(g * g, s, preferred_element_type=jnp.float32)
    mask = (lax.broadcasted_iota(jnp.int32, (1, 128), 1) != K).astype(jnp.float32)
    fm = 0.5 * jnp.sum(mask * (sv * sv - sv2), axis=1, keepdims=True)
    fm = fm + lax.slice(sv, (0, K), (sv.shape[0], K + 1))
    h = jnp.maximum(_bdot(g, w0_ref[...]) + b0_ref[...], 0.0)
    h = jnp.maximum(_bdot(h, w1_ref[...]) + b1_ref[...], 0.0)
    h = jnp.maximum(_bdot(h, w2_ref[...]) + b2_ref[...], 0.0)
    dnn = _bdot(h, w3_ref[...]) + b3_ref[...]
    out_ref[...] = jax.nn.sigmoid(fm + dnn)


def _tc_head(g, sel, W0p, b0, W1, b1, W2, b2, W3, b3):
    d1 = W0p.shape[1]
    d2 = W1.shape[1]
    d3 = W2.shape[1]
    return pl.pallas_call(
        _tc_body,
        grid=(B // BB,),
        in_specs=[
            pl.BlockSpec((BB, FP * KP), lambda i: (i, 0)),
            pl.BlockSpec((FP * KP, 128), lambda i: (0, 0)),
            pl.BlockSpec((FP * KP, d1), lambda i: (0, 0)),
            pl.BlockSpec((1, d1), lambda i: (0, 0)),
            pl.BlockSpec((d1, d2), lambda i: (0, 0)),
            pl.BlockSpec((1, d2), lambda i: (0, 0)),
            pl.BlockSpec((d2, d3), lambda i: (0, 0)),
            pl.BlockSpec((1, d3), lambda i: (0, 0)),
            pl.BlockSpec((d3, 1), lambda i: (0, 0)),
            pl.BlockSpec((1, 1), lambda i: (0, 0)),
        ],
        out_specs=pl.BlockSpec((BB, 1), lambda i: (i, 0)),
        out_shape=jax.ShapeDtypeStruct((B, 1), jnp.float32),
    )(g, sel, W0p, b0.reshape(1, -1), W1, b1.reshape(1, -1),
      W2, b2.reshape(1, -1), W3, b3.reshape(1, -1))


def kernel(feature, w, v, W0, b0, W1, b1, W2, b2, W3, b3):
    feat40 = jnp.concatenate(
        [feature, jnp.zeros((B, FP - F), feature.dtype)], axis=1)
    idx = _permute_idx(feat40.reshape(-1))
    tab = _pack_table(v, w)
    rows = _make_sc_gather()(idx, tab)
    g = rows.reshape(B, FP * KP)
    jj = jnp.arange(FP * KP)
    sel = ((jj[:, None] % KP == jnp.arange(128)[None, :]) & (jj[:, None] < F * KP)
           ).astype(jnp.float32)
    j = jnp.arange(F * K)
    W0p = jnp.zeros((FP * KP, W0.shape[1]), jnp.float32
                    ).at[(j // K) * KP + (j % K)].set(W0)
    out = _tc_head(g, sel, W0p, b0, W1, b1, W2, b2, W3, b3)
    return out.reshape(-1)


# speedup vs baseline: 1.3190x; 1.3190x over previous
import functools

import jax
import jax.numpy as jnp
from jax import lax
from jax.experimental import pallas as pl
from jax.experimental.pallas import tpu as pltpu
from jax.experimental.pallas import tpu_sc as plsc

F = 39
FP = 40
K = 10
KP = 16
B = 16384
TOTAL = B * FP
NC, NS = 2, 16
NW = NC * NS
PER_W = TOTAL // NW
NPL = 5
SPW = B // NW
CH = PER_W // NPL


@functools.cache
def _make_sc_gather():
    mesh = plsc.VectorSubcoreMesh(core_axis_name="c", subcore_axis_name="s")

    @functools.partial(
        pl.kernel,
        mesh=mesh,
        out_type=[jax.ShapeDtypeStruct((B * 8, KP), jnp.float32)
                  for _ in range(NPL)],
        scratch_types=[
            pltpu.VMEM((CH,), jnp.int32),
            pltpu.VMEM((CH, KP), jnp.float32),
            pltpu.SemaphoreType.DMA,
        ],
        compiler_params=pltpu.CompilerParams(use_tc_tiling_on_sc=False),
    )
    def _sc_gather(idx_hbm, tab_hbm, *out_and_scratch):
        outs = out_and_scratch[:NPL]
        idx_v, rows_v, sem = out_and_scratch[NPL:]
        wid = lax.axis_index("s") * NC + lax.axis_index("c")
        base = wid * PER_W
        sbase = wid * SPW
        for p in range(NPL):
            off = base + p * CH
            pltpu.sync_copy(idx_hbm.at[pl.ds(off, CH)], idx_v)
            pltpu.async_copy(tab_hbm.at[idx_v], rows_v, sem).wait()
            pltpu.sync_copy(rows_v, outs[p].at[pl.ds(sbase * 8, CH)])

    return _sc_gather


CN = 65536
CN8 = CN // 8
SH = CN8.bit_length() - 1


def _pack_body(vt_ref, wt_ref, out_ref):
    vt = vt_ref[...]
    wt = wt_ref[...]
    z = jnp.zeros((KP - K - 1, CN), jnp.float32)
    m = jnp.concatenate([vt, wt, z], axis=0)
    for q in range(CN // 1024):
        mq = jnp.concatenate(
            [m[:, CN8 * k + 128 * q: CN8 * k + 128 * q + 128]
             for k in range(8)], axis=0)
        out_ref[128 * q:128 * (q + 1), :] = mq.T


def _permute_idx(idx):
    o = idx & (CN - 1)
    return (idx & ~(CN - 1)) | ((o & (CN8 - 1)) << 3) | (o >> SH)


def _pack_table(v, w):
    fea = v.shape[0]
    grid = (fea + CN - 1) // CN
    out2d = pl.pallas_call(
        _pack_body,
        grid=(grid,),
        in_specs=[
            pl.BlockSpec((K, CN), lambda i: (0, i)),
            pl.BlockSpec((1, CN), lambda i: (0, i)),
        ],
        out_specs=pl.BlockSpec((CN * KP // 128, 128), lambda i: (i, 0)),
        out_shape=jax.ShapeDtypeStruct((grid * CN * KP // 128, 128), jnp.float32),
    )(v.T, w.T)
    return out2d.reshape(grid * CN, KP)


BB = 512


def _bdot(a, b):
    return jnp.dot(a.astype(jnp.bfloat16), b.astype(jnp.bfloat16),
                   preferred_element_type=jnp.float32)


def _tc_body(g0_ref, g1_ref, g2_ref, g3_ref, g4_ref, s_ref, w0_ref, b0_ref,
             w1_ref, b1_ref, w2_ref, b2_ref, w3_ref, b3_ref, out_ref):
    g = jnp.concatenate(
        [g0_ref[...], g1_ref[...], g2_ref[...], g3_ref[...], g4_ref[...]],
        axis=1)
    s = s_ref[...]
    sv = jnp.dot(g, s, preferred_element_type=jnp.float32)
    sv2 = jnp.dot(g * g, s, preferred_element_type=jnp.float32)
    mask = (lax.broadcasted_iota(jnp.int32, (1, 128), 1) != K).astype(jnp.float32)
    fm = 0.5 * jnp.sum(mask * (sv * sv - sv2), axis=1, keepdims=True)
    fm = fm + lax.slice(sv, (0, K), (sv.shape[0], K + 1))
    h = jnp.maximum(_bdot(g, w0_ref[...]) + b0_ref[...], 0.0)
    h = jnp.maximum(_bdot(h, w1_ref[...]) + b1_ref[...], 0.0)
    h = jnp.maximum(_bdot(h, w2_ref[...]) + b2_ref[...], 0.0)
    dnn = _bdot(h, w3_ref[...]) + b3_ref[...]
    out_ref[...] = jax.nn.sigmoid(fm + dnn)


def _tc_head(gs, sel, W0p, b0, W1, b1, W2, b2, W3, b3):
    d1 = W0p.shape[1]
    d2 = W1.shape[1]
    d3 = W2.shape[1]
    return pl.pallas_call(
        _tc_body,
        grid=(B // BB,),
        in_specs=[
            *[pl.BlockSpec((BB, 128), lambda i: (i, 0)) for _ in range(NPL)],
            pl.BlockSpec((FP * KP, 128), lambda i: (0, 0)),
            pl.BlockSpec((FP * KP, d1), lambda i: (0, 0)),
            pl.BlockSpec((1, d1), lambda i: (0, 0)),
            pl.BlockSpec((d1, d2), lambda i: (0, 0)),
            pl.BlockSpec((1, d2), lambda i: (0, 0)),
            pl.BlockSpec((d2, d3), lambda i: (0, 0)),
            pl.BlockSpec((1, d3), lambda i: (0, 0)),
            pl.BlockSpec((d3, 1), lambda i: (0, 0)),
            pl.BlockSpec((1, 1), lambda i: (0, 0)),
        ],
        out_specs=pl.BlockSpec((BB, 1), lambda i: (i, 0)),
        out_shape=jax.ShapeDtypeStruct((B, 1), jnp.float32),
    )(*gs, sel, W0p, b0.reshape(1, -1), W1, b1.reshape(1, -1),
      W2, b2.reshape(1, -1), W3, b3.reshape(1, -1))


def kernel(feature, w, v, W0, b0, W1, b1, W2, b2, W3, b3):
    feat40 = jnp.concatenate(
        [feature, (jnp.arange(B, dtype=feature.dtype) % v.shape[0]
                   ).reshape(B, 1)], axis=1)
    idx = _permute_idx(
        feat40.reshape(NW, SPW, NPL, 8).transpose(0, 2, 1, 3).reshape(-1))
    tab = _pack_table(v, w)
    gs = [r.reshape(B, 128) for r in _make_sc_gather()(idx, tab)]
    jj = jnp.arange(FP * KP)
    sel = ((jj[:, None] % KP == jnp.arange(128)[None, :]) & (jj[:, None] < F * KP)
           ).astype(jnp.float32)
    j = jnp.arange(F * K)
    W0p = jnp.zeros((FP * KP, W0.shape[1]), jnp.float32
                    ).at[(j // K) * KP + (j % K)].set(W0)
    out = _tc_head(gs, sel, W0p, b0, W1, b1, W2, b2, W3, b3)
    return out.reshape(-1)
